# Initial kernel scaffold; baseline (speedup 1.0000x reference)
import jax
import jax.numpy as jnp
from jax.experimental import pallas as pl


def kernel(x, spiral_indices, W1, b1, W2, b2):
    raise NotImplementedError("write your pallas kernel here")


# speedup vs baseline: 3.4686x; 3.4686x over previous
import functools

import jax
import jax.numpy as jnp
from jax import lax
from jax.experimental import pallas as pl
from jax.experimental.pallas import tpu as pltpu
from jax.experimental.pallas import tpu_sc as plsc

_NC = 2
_NS = 16
_NW = _NC * _NS
_L = 16

_SEQ = 16
_CH = 8


def _mm_body(a_ref, w_ref, o_ref):
    o_ref[...] = jnp.dot(a_ref[...], w_ref[...], preferred_element_type=jnp.float32)


def _matmul_tc(a, w, block_rows):
    m, k = a.shape
    _, n = w.shape
    return pl.pallas_call(
        _mm_body,
        grid=(m // block_rows,),
        in_specs=[
            pl.BlockSpec((block_rows, k), lambda i: (i, 0)),
            pl.BlockSpec((k, n), lambda i: (0, 0)),
        ],
        out_specs=pl.BlockSpec((block_rows, n), lambda i: (i, 0)),
        out_shape=jax.ShapeDtypeStruct((m, n), jnp.float32),
    )(a, w)


def _make_gather_reduce(np_nodes, c, apply_elu):
    pt = np_nodes // _NW
    nchunk = pt // _CH
    rows = _CH * _SEQ
    cvec = c // _L

    mesh = plsc.VectorSubcoreMesh(
        core_axis_name="c", subcore_axis_name="s", num_cores=_NC, num_subcores=_NS
    )

    @functools.partial(
        pl.kernel,
        out_type=jax.ShapeDtypeStruct((np_nodes, c), jnp.float32),
        mesh=mesh,
        compiler_params=pltpu.CompilerParams(use_tc_tiling_on_sc=False),
        scratch_types=[
            pltpu.VMEM((pt * _SEQ,), jnp.int32),
            pltpu.VMEM((rows, c), jnp.float32),
            pltpu.VMEM((_CH, c), jnp.float32),
            pltpu.VMEM((c,), jnp.float32),
            pltpu.SemaphoreType.DMA,
        ],
    )
    def body(table_hbm, idx_hbm, bias_hbm, out_hbm, idxv, gbuf, obuf, biasv, sem):
        wid = lax.axis_index("s") * _NC + lax.axis_index("c")
        base = wid * pt

        pltpu.sync_copy(idx_hbm.at[pl.ds(base * _SEQ, pt * _SEQ)], idxv)
        pltpu.sync_copy(bias_hbm, biasv)

        lane = lax.broadcasted_iota(jnp.int32, (_L,), 0)

        @pl.loop(0, pt * _SEQ // _L)
        def _flatten(i):
            off = pl.multiple_of(i * _L, _L)
            v = idxv[pl.ds(off, _L)]
            idxv[pl.ds(off, _L)] = v * _SEQ + lane

        bias_vecs = [biasv[pl.ds(j * _L, _L)] for j in range(cvec)]

        @pl.loop(0, nchunk)
        def _chunk(g):
            roff = pl.multiple_of(g * rows, rows)
            pltpu.async_copy(
                table_hbm.at[idxv.at[pl.ds(roff, rows)]], gbuf, sem
            ).wait()
            for nloc in range(_CH):
                acc = list(bias_vecs)
                for s in range(_SEQ):
                    r = nloc * _SEQ + s
                    for j in range(cvec):
                        acc[j] = acc[j] + gbuf[r, pl.ds(j * _L, _L)]
                for j in range(cvec):
                    v = acc[j]
                    if apply_elu:
                        v = jnp.where(v > 0.0, v, jnp.exp(v) - 1.0)
                    obuf[nloc, pl.ds(j * _L, _L)] = v
            pltpu.sync_copy(obuf, out_hbm.at[pl.ds(base + g * _CH, _CH)])

    return body


def kernel(x, spiral_indices, W1, b1, W2, b2):
    n = x.shape[0]
    c0 = x.shape[1]
    c1 = W1.shape[1]
    c2 = W2.shape[1]

    blk = 1024
    np_nodes = ((n + blk - 1) // blk) * blk

    h0 = jnp.pad(x[:, :, 0], ((0, np_nodes - n), (0, 0)))
    idx_flat = jnp.pad(spiral_indices, ((0, np_nodes - n), (0, 0))).reshape(-1)

    w1c = W1.reshape(_SEQ, c0, c1).transpose(1, 0, 2).reshape(c0, _SEQ * c1)
    w2c = W2.reshape(_SEQ, c1, c2).transpose(1, 0, 2).reshape(c1, _SEQ * c2)

    y1 = _matmul_tc(h0, w1c, blk)
    g1 = _make_gather_reduce(np_nodes, c1, True)
    h1 = g1(y1.reshape(np_nodes * _SEQ, c1), idx_flat, b1)

    y2 = _matmul_tc(h1, w2c, blk)
    g2 = _make_gather_reduce(np_nodes, c2, False)
    out = g2(y2.reshape(np_nodes * _SEQ, c2), idx_flat, b2)

    return out[:n, :, None]


# speedup vs baseline: 4.2541x; 1.2265x over previous
import functools

import jax
import jax.numpy as jnp
from jax import lax
from jax.experimental import pallas as pl
from jax.experimental.pallas import tpu as pltpu
from jax.experimental.pallas import tpu_sc as plsc

_NC = 2
_NS = 16
_NW = _NC * _NS
_L = 16

_SEQ = 16
_CH = 8


def _mm_body(a_ref, w_ref, o_ref):
    o_ref[...] = jnp.dot(a_ref[...], w_ref[...], preferred_element_type=jnp.float32)


def _matmul_tc(a, w, block_rows):
    m, k = a.shape
    _, n = w.shape
    return pl.pallas_call(
        _mm_body,
        grid=(m // block_rows,),
        in_specs=[
            pl.BlockSpec((block_rows, k), lambda i: (i, 0)),
            pl.BlockSpec((k, n), lambda i: (0, 0)),
        ],
        out_specs=pl.BlockSpec((block_rows, n), lambda i: (i, 0)),
        out_shape=jax.ShapeDtypeStruct((m, n), jnp.float32),
    )(a, w)


def _make_gather_reduce(np_nodes, c, apply_elu):
    pt = np_nodes // _NW
    nchunk = pt // _CH
    rows = _CH * _SEQ
    cvec = c // _L

    mesh = plsc.VectorSubcoreMesh(
        core_axis_name="c", subcore_axis_name="s", num_cores=_NC, num_subcores=_NS
    )

    @functools.partial(
        pl.kernel,
        out_type=jax.ShapeDtypeStruct((np_nodes, c), jnp.float32),
        mesh=mesh,
        compiler_params=pltpu.CompilerParams(use_tc_tiling_on_sc=False),
        scratch_types=[
            pltpu.VMEM((pt * _SEQ,), jnp.int32),
            pltpu.VMEM((rows, c), jnp.float32),
            pltpu.VMEM((rows, c), jnp.float32),
            pltpu.VMEM((_CH, c), jnp.float32),
            pltpu.VMEM((_CH, c), jnp.float32),
            pltpu.VMEM((c,), jnp.float32),
            pltpu.SemaphoreType.DMA,
            pltpu.SemaphoreType.DMA,
            pltpu.SemaphoreType.DMA,
            pltpu.SemaphoreType.DMA,
        ],
    )
    def body(table_hbm, idx_hbm, bias_hbm, out_hbm,
             idxv, gbuf0, gbuf1, obuf0, obuf1, biasv,
             gsem0, gsem1, osem0, osem1):
        wid = lax.axis_index("s") * _NC + lax.axis_index("c")
        base = wid * pt

        pltpu.sync_copy(idx_hbm.at[pl.ds(base * _SEQ, pt * _SEQ)], idxv)
        pltpu.sync_copy(bias_hbm, biasv)

        lane = lax.broadcasted_iota(jnp.int32, (_L,), 0)

        @pl.loop(0, pt * _SEQ // _L, unroll=4)
        def _flatten(i):
            off = pl.multiple_of(i * _L, _L)
            v = idxv[pl.ds(off, _L)]
            idxv[pl.ds(off, _L)] = v * _SEQ + lane

        bias_vecs = [biasv[pl.ds(j * _L, _L)] for j in range(cvec)]

        def fire(g, gbuf, gsem):
            roff = pl.multiple_of(g * rows, rows)
            pltpu.async_copy(table_hbm.at[idxv.at[pl.ds(roff, rows)]], gbuf, gsem)

        def reduce_chunk(g, gbuf, gsem, obuf, osem):
            pltpu.make_async_copy(table_hbm.at[pl.ds(0, rows)], gbuf, gsem).wait()

            @pl.when(g >= 2)
            def _():
                pltpu.make_async_copy(out_hbm.at[pl.ds(0, _CH)], obuf, osem).wait()
            for nloc in range(_CH):
                acc = list(bias_vecs)
                for s in range(_SEQ):
                    r = nloc * _SEQ + s
                    for j in range(cvec):
                        acc[j] = acc[j] + gbuf[r, pl.ds(j * _L, _L)]
                for j in range(cvec):
                    v = acc[j]
                    if apply_elu:
                        v = jnp.where(v > 0.0, v, jnp.exp(v) - 1.0)
                    obuf[nloc, pl.ds(j * _L, _L)] = v
            @pl.when(g + 2 < nchunk)
            def _():
                fire(g + 2, gbuf, gsem)
            pltpu.async_copy(obuf, out_hbm.at[pl.ds(base + g * _CH, _CH)], osem)

        fire(0, gbuf0, gsem0)
        fire(1, gbuf1, gsem1)

        @pl.loop(0, nchunk // 2)
        def _pair(h):
            g0 = pl.multiple_of(h * 2, 2)
            reduce_chunk(g0, gbuf0, gsem0, obuf0, osem0)
            reduce_chunk(g0 + 1, gbuf1, gsem1, obuf1, osem1)

        pltpu.make_async_copy(out_hbm.at[pl.ds(0, _CH)], obuf0, osem0).wait()
        pltpu.make_async_copy(out_hbm.at[pl.ds(0, _CH)], obuf1, osem1).wait()

    return body


def kernel(x, spiral_indices, W1, b1, W2, b2):
    n = x.shape[0]
    c0 = x.shape[1]
    c1 = W1.shape[1]
    c2 = W2.shape[1]

    blk = 1024
    np_nodes = ((n + blk - 1) // blk) * blk

    h0 = jnp.pad(x[:, :, 0], ((0, np_nodes - n), (0, 0)))
    idx_flat = jnp.pad(spiral_indices, ((0, np_nodes - n), (0, 0))).reshape(-1)

    w1c = W1.reshape(_SEQ, c0, c1).transpose(1, 0, 2).reshape(c0, _SEQ * c1)
    w2c = W2.reshape(_SEQ, c1, c2).transpose(1, 0, 2).reshape(c1, _SEQ * c2)

    y1 = _matmul_tc(h0, w1c, blk)
    g1 = _make_gather_reduce(np_nodes, c1, True)
    h1 = g1(y1.reshape(np_nodes * _SEQ, c1), idx_flat, b1)

    y2 = _matmul_tc(h1, w2c, blk)
    g2 = _make_gather_reduce(np_nodes, c2, False)
    out = g2(y2.reshape(np_nodes * _SEQ, c2), idx_flat, b2)

    return out[:n, :, None]
